# Initial kernel scaffold; baseline (speedup 1.0000x reference)
import jax
import jax.numpy as jnp
from jax.experimental import pallas as pl


def kernel(x, edge_index, Wlin, blin, att, bias_out, gamma, beta):
    raise NotImplementedError("write your pallas kernel here")


# speedup vs baseline: 1.1114x; 1.1114x over previous
import functools
import jax
import jax.numpy as jnp
from jax.experimental import pallas as pl
from jax.experimental.pallas import tpu as pltpu

B, C_IN = 2, 128
D, H, W = 10, 25, 20
C_OUT = 128
HEADS = 2
E = 160000
N = B * D * H * W
NEG_SLOPE_GAT = 0.2
NEG_SLOPE_ACT = 0.01
BN_EPS = 1e-5

DHW = D * H * W
PADW = 144


def _lrelu(v, s):
    return jnp.maximum(v, s * v)


def _mm_kernel(x_ref, w_ref, b_ref, o_ref):
    o_ref[0] = (
        jnp.dot(x_ref[...], w_ref[...], preferred_element_type=jnp.float32)
        + b_ref[0]
    )


def _linear(xf, Wlin, blin):
    blk = 2000
    grid = (HEADS, N // blk)
    return pl.pallas_call(
        _mm_kernel,
        grid=grid,
        in_specs=[
            pl.BlockSpec((blk, C_IN), lambda h, i: (i, 0)),
            pl.BlockSpec((C_IN, C_OUT), lambda h, i: (0, h)),
            pl.BlockSpec((1, 1, C_OUT), lambda h, i: (h, 0, 0)),
        ],
        out_specs=pl.BlockSpec((1, blk, C_OUT), lambda h, i: (h, i, 0)),
        out_shape=jax.ShapeDtypeStruct((HEADS, N, C_OUT), jnp.float32),
    )(xf, Wlin, blin.reshape(HEADS, 1, C_OUT))


def _nf_kernel(u_ref, bias_ref, o_ref):
    u0 = u_ref[0, :, :C_OUT]
    u1 = u_ref[1, :, :C_OUT]
    d0 = jnp.sum(u_ref[0, :, C_OUT:], axis=-1)[:, None] + 1e-16
    d1 = jnp.sum(u_ref[1, :, C_OUT:], axis=-1)[:, None] + 1e-16
    o_ref[...] = 0.5 * (u0 / d0 + u1 / d1) + bias_ref[...]


def _bn_kernel(a_ref, g_ref, b_ref, o_ref):
    a = a_ref[...]
    m = jnp.mean(a, axis=1, keepdims=True)
    ex2 = jnp.mean(a * a, axis=1, keepdims=True)
    mc = 0.5 * (m[:C_OUT] + m[C_OUT:])
    ex2c = 0.5 * (ex2[:C_OUT] + ex2[C_OUT:])
    var = ex2c - mc * mc
    scale = jax.lax.rsqrt(var + BN_EPS) * g_ref[...].reshape(C_OUT, 1)
    shift = b_ref[...].reshape(C_OUT, 1) - mc * scale
    scale2 = jnp.concatenate([scale, scale], axis=0)
    shift2 = jnp.concatenate([shift, shift], axis=0)
    y = a * scale2 + shift2
    o_ref[...] = jnp.maximum(y, NEG_SLOPE_ACT * y)


def _bn_tail(U, bias_out, gamma, beta):
    nf = pl.pallas_call(
        _nf_kernel,
        out_shape=jax.ShapeDtypeStruct((N, C_OUT), jnp.float32),
    )(U, bias_out.reshape(1, C_OUT))
    a = nf.reshape(B * C_OUT, DHW)
    out = pl.pallas_call(
        _bn_kernel,
        out_shape=jax.ShapeDtypeStruct((B * C_OUT, DHW), jnp.float32),
    )(a, gamma.reshape(1, C_OUT), beta.reshape(1, C_OUT))
    return out


def _idn(x_ref, o_ref):
    o_ref[...] = x_ref[...]


def kernel(x, edge_index, Wlin, blin, att, bias_out, gamma, beta):
    xf = jnp.transpose(x.reshape(B, C_IN, DHW), (0, 2, 1)).reshape(-1, C_IN)
    hh = _linear(xf, Wlin, blin)
    h = hh.swapaxes(0, 1)
    src = edge_index[0]; dst = edge_index[1]
    loop = jnp.arange(N, dtype=src.dtype)
    src2 = jnp.concatenate([src, loop]); dst2 = jnp.concatenate([dst, loop])
    valid = jnp.concatenate([src != dst, jnp.ones((N,), dtype=bool)])
    hs = h[src2]; hd = h[dst2]
    e = _lrelu(hs + hd, NEG_SLOPE_GAT)
    scores = (e * att[None]).sum(-1)
    g = jnp.max(scores, axis=0, keepdims=True)
    q = jnp.where(valid[:, None], jnp.exp(scores - g), 0.0)
    denom = jax.ops.segment_sum(q, dst2, num_segments=N)
    msg = hs * q[..., None]
    Um = jax.ops.segment_sum(msg, dst2, num_segments=N)
    U = jnp.concatenate(
        [Um.swapaxes(0, 1), denom.T[..., None],
         jnp.zeros((HEADS, N, PADW - C_OUT - 1), jnp.float32)], axis=-1)
    out = _bn_tail(U, bias_out, gamma, beta)
    return out.reshape(B, C_OUT, D, H, W)


# speedup vs baseline: 10.4826x; 9.4321x over previous
import functools
import jax
import jax.numpy as jnp
from jax import lax
from jax.experimental import pallas as pl
from jax.experimental.pallas import tpu as pltpu
from jax.experimental.pallas import tpu_sc as plsc

B, C_IN = 2, 128
D, H, W = 10, 25, 20
C_OUT = 128
HEADS = 2
E = 160000
N = B * D * H * W
NEG_SLOPE_GAT = 0.2
NEG_SLOPE_ACT = 0.01
BN_EPS = 1e-5

DHW = D * H * W
PADW = 144
ETOT = E + N
NTILE = 16
BLK = 64
GPB = BLK // 16
NBLK = 168
EPT = NBLK * BLK
EPAD = NTILE * EPT
NPAD = 10112
NPT = NPAD // NTILE


def _lrelu(v, s):
    return jnp.maximum(v, s * v)


def _mm_kernel(x_ref, w_ref, b_ref, o_ref):
    o_ref[0] = (
        jnp.dot(x_ref[...], w_ref[...], preferred_element_type=jnp.float32)
        + b_ref[0]
    )


def _linear(xf, Wlin, blin):
    blk = 2000
    grid = (HEADS, N // blk)
    return pl.pallas_call(
        _mm_kernel,
        grid=grid,
        in_specs=[
            pl.BlockSpec((blk, C_IN), lambda h, i: (i, 0)),
            pl.BlockSpec((C_IN, C_OUT), lambda h, i: (0, h)),
            pl.BlockSpec((1, 1, C_OUT), lambda h, i: (h, 0, 0)),
        ],
        out_specs=pl.BlockSpec((1, blk, C_OUT), lambda h, i: (h, i, 0)),
        out_shape=jax.ShapeDtypeStruct((HEADS, N, C_OUT), jnp.float32),
    )(xf, Wlin, blin.reshape(HEADS, 1, C_OUT))


NHALF = 5056
NROWS = 5120
NDEN = 640
NACCH = NROWS + NDEN
NPTH = NROWS // NTILE
NDT = NDEN // NTILE


def _sc_edge(h2, src2p, dst2p, attf):
    mesh = plsc.VectorSubcoreMesh(core_axis_name="c", subcore_axis_name="s")

    @functools.partial(
        pl.kernel,
        out_type=jax.ShapeDtypeStruct((HEADS * 2 * NACCH, C_OUT), jnp.float32),
        mesh=mesh,
        scratch_types=[
            pltpu.VMEM((BLK,), jnp.int32),
            pltpu.VMEM((BLK,), jnp.int32),
            pltpu.VMEM((BLK,), jnp.int32),
            pltpu.VMEM((BLK,), jnp.int32),
            pltpu.VMEM((BLK,), jnp.int32),
            pltpu.VMEM((BLK, C_OUT), jnp.float32),
            pltpu.VMEM((BLK, C_OUT), jnp.float32),
            pltpu.VMEM((BLK, C_OUT), jnp.float32),
            pltpu.VMEM((BLK, C_OUT), jnp.float32),
            pltpu.VMEM((EPT,), jnp.float32),
            pltpu.VMEM((C_OUT,), jnp.float32),
            pltpu.VMEM((8, 16), jnp.float32),
            pltpu.VMEM((NTILE * 8, 16), jnp.float32),
            pltpu.VMEM_SHARED((NACCH, C_OUT), jnp.float32),
            pltpu.VMEM_SHARED((NTILE * 8, 16), jnp.float32),
            pltpu.SemaphoreType.DMA,
            pltpu.SemaphoreType.DMA,
        ],
    )
    def k(h2_hbm, src_hbm, dst_hbm, att_hbm, u_hbm,
          sidx, didx, didx2, gsidx, gdidx, srows, drows, msg, dmsg, sbuf,
          attv, mbuf, gred, ush, gsh, sem1, sem2):
        head = lax.axis_index("c")
        sid = lax.axis_index("s")
        base_e = sid * EPT
        hoff = head * N
        iota16 = lax.iota(jnp.int32, 16)

        pltpu.sync_copy(att_hbm.at[pl.ds(head * C_OUT, C_OUT)], attv)

        def zero_buffers():
            def zero_row(r, _):
                for j in range(C_OUT // 16):
                    msg[r, pl.ds(j * 16, 16)] = jnp.zeros((16,), jnp.float32)
                    dmsg[r, pl.ds(j * 16, 16)] = jnp.zeros((16,), jnp.float32)
                return 0
            lax.fori_loop(0, BLK, zero_row, 0)

        def zero_acc_slices():
            for kk in range(5):
                pltpu.sync_copy(msg.at[pl.ds(0, 64)],
                                ush.at[pl.ds(sid * NPTH + kk * 64, 64)])
            pltpu.sync_copy(msg.at[pl.ds(0, NDT)],
                            ush.at[pl.ds(NROWS + sid * NDT, NDT)])

        zero_buffers()
        zero_acc_slices()

        def blk_body(blk_i, macc):
            eb = base_e + blk_i * BLK
            pltpu.sync_copy(src_hbm.at[pl.ds(eb, BLK)], sidx)
            pltpu.sync_copy(dst_hbm.at[pl.ds(eb, BLK)], didx)
            for j in range(GPB):
                gsidx[pl.ds(j * 16, 16)] = sidx[pl.ds(j * 16, 16)] + hoff
                gdidx[pl.ds(j * 16, 16)] = didx[pl.ds(j * 16, 16)] + hoff
            cp1 = pltpu.async_copy(h2_hbm.at[gsidx], srows, sem1)
            cp2 = pltpu.async_copy(h2_hbm.at[gdidx], drows, sem2)
            cp1.wait()
            cp2.wait()
            attg = [attv[pl.ds(cg * 16, 16)] for cg in range(8)]

            def g_body(g, mg):
                sc = jnp.zeros((16,), jnp.float32)
                for jj in range(16):
                    e = g * 16 + jj
                    acc = jnp.zeros((16,), jnp.float32)
                    for cg in range(8):
                        v = (srows[e, pl.ds(cg * 16, 16)]
                             + drows[e, pl.ds(cg * 16, 16)])
                        acc = acc + attg[cg] * jnp.maximum(v, NEG_SLOPE_GAT * v)
                    for kk in (8, 4, 2, 1):
                        acc = acc + acc[iota16 ^ kk]
                    sc = jnp.where(iota16 == jj, acc, sc)
                sbuf[pl.ds(blk_i * BLK + g * 16, 16)] = sc
                return jnp.maximum(mg, sc)

            return lax.fori_loop(0, GPB, g_body, macc)

        macc = lax.fori_loop(0, NBLK, blk_body,
                             jnp.full((16,), -1e30, jnp.float32))

        for r in range(8):
            mbuf[r, :] = macc
        pltpu.sync_copy(mbuf, gsh.at[pl.ds(sid * 8, 8)])
        plsc.subcore_barrier()
        pltpu.sync_copy(gsh, gred)
        gv = gred[0, :]
        for r in range(1, NTILE):
            gv = jnp.maximum(gv, gred[r * 8, :])
        for kk in (8, 4, 2, 1):
            gv = jnp.maximum(gv, gv[iota16 ^ kk])
        gmax = gv
        plsc.subcore_barrier()

        for half in range(2):
            lo = half * NHALF

            def mb_body(blk_i, carry):
                eb = base_e + blk_i * BLK
                pltpu.sync_copy(src_hbm.at[pl.ds(eb, BLK)], sidx)
                pltpu.sync_copy(dst_hbm.at[pl.ds(eb, BLK)], gdidx)
                for j in range(GPB):
                    gsidx[pl.ds(j * 16, 16)] = sidx[pl.ds(j * 16, 16)] + hoff
                    dl = jnp.clip(gdidx[pl.ds(j * 16, 16)] - lo, 0, NHALF - 1)
                    didx[pl.ds(j * 16, 16)] = dl
                    didx2[pl.ds(j * 16, 16)] = (
                        NROWS + lax.shift_right_logical(dl, 3))
                pltpu.async_copy(h2_hbm.at[gsidx], srows, sem1).wait()
                zero16 = jnp.zeros((16,), jnp.float32)

                def mg_body(g, cg2):
                    s16 = sbuf[pl.ds(blk_i * BLK + g * 16, 16)]
                    sv = sidx[pl.ds(g * 16, 16)]
                    dv = gdidx[pl.ds(g * 16, 16)]
                    eg = jnp.full((16,), eb + g * 16, jnp.int32) + iota16
                    valid = jnp.logical_and(
                        eg < ETOT, jnp.logical_or(sv != dv, eg >= E))
                    valid = jnp.logical_and(
                        valid, jnp.logical_and(dv >= lo, dv < lo + NHALF))
                    q16 = jnp.where(valid, jnp.exp(s16 - gmax), 0.0)
                    dloc = dv - lo
                    for jj in range(16):
                        e = g * 16 + jj
                        qsplat = jnp.full((16,), q16[jj], jnp.float32)
                        dlane = jnp.full((16,), dloc[jj] & 7, jnp.int32)
                        for cg in range(8):
                            msg[e, pl.ds(cg * 16, 16)] = (
                                srows[e, pl.ds(cg * 16, 16)] * qsplat)
                        dmsg[e, pl.ds(0, 16)] = jnp.where(
                            iota16 == dlane, qsplat, zero16)
                    return cg2

                lax.fori_loop(0, GPB, mg_body, 0)
                pltpu.sync_copy(msg, ush.at[didx], add=True)
                pltpu.sync_copy(dmsg, ush.at[didx2], add=True)
                return carry

            lax.fori_loop(0, NBLK, mb_body, 0)

            plsc.subcore_barrier()
            uoff = (head * 2 + half) * NACCH
            pltpu.sync_copy(ush.at[pl.ds(sid * NPTH, NPTH)],
                            u_hbm.at[pl.ds(uoff + sid * NPTH, NPTH)])
            pltpu.sync_copy(ush.at[pl.ds(NROWS + sid * NDT, NDT)],
                            u_hbm.at[pl.ds(uoff + NROWS + sid * NDT, NDT)])
            if half == 0:
                plsc.subcore_barrier()
                zero_buffers()
                zero_acc_slices()
                plsc.subcore_barrier()

    return k(h2, src2p, dst2p, attf)


def _nf_kernel(u_ref, d_ref, bias_ref, o_ref):
    u0 = u_ref[0]
    u1 = u_ref[1]
    d0 = d_ref[0] + 1e-16
    d1 = d_ref[1] + 1e-16
    o_ref[...] = 0.5 * (u0 / d0 + u1 / d1) + bias_ref[...]


def _bn_kernel(a_ref, g_ref, b_ref, o_ref):
    a = a_ref[...]
    m = jnp.mean(a, axis=1, keepdims=True)
    ex2 = jnp.mean(a * a, axis=1, keepdims=True)
    mc = 0.5 * (m[:C_OUT] + m[C_OUT:])
    ex2c = 0.5 * (ex2[:C_OUT] + ex2[C_OUT:])
    var = ex2c - mc * mc
    scale = lax.rsqrt(var + BN_EPS) * g_ref[...].reshape(C_OUT, 1)
    shift = b_ref[...].reshape(C_OUT, 1) - mc * scale
    scale2 = jnp.concatenate([scale, scale], axis=0)
    shift2 = jnp.concatenate([shift, shift], axis=0)
    y = a * scale2 + shift2
    o_ref[...] = jnp.maximum(y, NEG_SLOPE_ACT * y)


def _bn_tail(U, den, bias_out, gamma, beta):
    nf = pl.pallas_call(
        _nf_kernel,
        out_shape=jax.ShapeDtypeStruct((N, C_OUT), jnp.float32),
    )(U, den, bias_out.reshape(1, C_OUT))
    a = nf.reshape(B * C_OUT, DHW)
    return pl.pallas_call(
        _bn_kernel,
        out_shape=jax.ShapeDtypeStruct((B * C_OUT, DHW), jnp.float32),
    )(a, gamma.reshape(1, C_OUT), beta.reshape(1, C_OUT))


def kernel(x, edge_index, Wlin, blin, att, bias_out, gamma, beta):
    xf = jnp.transpose(x.reshape(B, C_IN, DHW), (0, 2, 1)).reshape(-1, C_IN)
    h2 = _linear(xf, Wlin, blin).reshape(HEADS * N, C_OUT)

    src = edge_index[0]
    dst = edge_index[1]
    loop = jnp.arange(N, dtype=jnp.int32)
    padz = jnp.zeros((EPAD - ETOT,), jnp.int32)
    src2p = jnp.concatenate([src, loop, padz])
    dst2p = jnp.concatenate([dst, loop, padz])

    U = _sc_edge(h2, src2p, dst2p, att.reshape(HEADS * C_OUT))
    U4 = U.reshape(HEADS, 2, NACCH, C_OUT)
    Um = U4[:, :, :NHALF, :].reshape(HEADS, 2 * NHALF, C_OUT)[:, :N, :]
    den = U4[:, :, NROWS:NROWS + NHALF // 8, :8].reshape(
        HEADS, 2 * NHALF)[:, :N].reshape(HEADS, N, 1)
    out = _bn_tail(Um, den, bias_out, gamma, beta)
    return out.reshape(B, C_OUT, D, H, W)


# speedup vs baseline: 10.9078x; 1.0406x over previous
import functools
import jax
import jax.numpy as jnp
from jax import lax
from jax.experimental import pallas as pl
from jax.experimental.pallas import tpu as pltpu
from jax.experimental.pallas import tpu_sc as plsc

B, C_IN = 2, 128
D, H, W = 10, 25, 20
C_OUT = 128
HEADS = 2
E = 160000
N = B * D * H * W
NEG_SLOPE_GAT = 0.2
NEG_SLOPE_ACT = 0.01
BN_EPS = 1e-5

DHW = D * H * W
PADW = 144
ETOT = E + N
NTILE = 16
BLK = 64
GPB = BLK // 16
NBLK = 168
EPT = NBLK * BLK
EPAD = NTILE * EPT
NPAD = 10112
NPT = NPAD // NTILE


def _lrelu(v, s):
    return jnp.maximum(v, s * v)


def _mm_kernel(x_ref, w_ref, b_ref, o_ref):
    o_ref[0] = (
        jnp.dot(x_ref[...], w_ref[...], preferred_element_type=jnp.float32)
        + b_ref[0]
    )


def _linear(xf, Wlin, blin):
    blk = 2000
    grid = (HEADS, N // blk)
    return pl.pallas_call(
        _mm_kernel,
        grid=grid,
        in_specs=[
            pl.BlockSpec((blk, C_IN), lambda h, i: (i, 0)),
            pl.BlockSpec((C_IN, C_OUT), lambda h, i: (0, h)),
            pl.BlockSpec((1, 1, C_OUT), lambda h, i: (h, 0, 0)),
        ],
        out_specs=pl.BlockSpec((1, blk, C_OUT), lambda h, i: (h, i, 0)),
        out_shape=jax.ShapeDtypeStruct((HEADS, N, C_OUT), jnp.float32),
    )(xf, Wlin, blin.reshape(HEADS, 1, C_OUT))


NHALF = 5056
NROWS = 5120
NDEN = 640
NACCH = NROWS + NDEN
NPTH = NROWS // NTILE
NDT = NDEN // NTILE


def _sc_edge(h2, src2p, dst2p, attf):
    mesh = plsc.VectorSubcoreMesh(core_axis_name="c", subcore_axis_name="s")

    @functools.partial(
        pl.kernel,
        out_type=jax.ShapeDtypeStruct((HEADS * 2 * NACCH, C_OUT), jnp.float32),
        mesh=mesh,
        scratch_types=[
            pltpu.VMEM((BLK,), jnp.int32),
            pltpu.VMEM((BLK,), jnp.int32),
            pltpu.VMEM((BLK,), jnp.int32),
            pltpu.VMEM((BLK,), jnp.int32),
            pltpu.VMEM((BLK,), jnp.int32),
            pltpu.VMEM((BLK, C_OUT), jnp.float32),
            pltpu.VMEM((BLK, C_OUT), jnp.float32),
            pltpu.VMEM((BLK, C_OUT), jnp.float32),
            pltpu.VMEM((BLK, C_OUT), jnp.float32),
            pltpu.VMEM((EPT,), jnp.float32),
            pltpu.VMEM((C_OUT,), jnp.float32),
            pltpu.VMEM((8, 16), jnp.float32),
            pltpu.VMEM((NTILE * 8, 16), jnp.float32),
            pltpu.VMEM_SHARED((NACCH, C_OUT), jnp.float32),
            pltpu.VMEM_SHARED((NTILE * 8, 16), jnp.float32),
            pltpu.SemaphoreType.DMA,
            pltpu.SemaphoreType.DMA,
        ],
    )
    def k(h2_hbm, src_hbm, dst_hbm, att_hbm, u_hbm,
          sidx, didx, didx2, gsidx, gdidx, srows, drows, msg, dmsg, sbuf,
          attv, mbuf, gred, ush, gsh, sem1, sem2):
        head = lax.axis_index("c")
        sid = lax.axis_index("s")
        base_e = sid * EPT
        hoff = head * N
        iota16 = lax.iota(jnp.int32, 16)

        pltpu.sync_copy(att_hbm.at[pl.ds(head * C_OUT, C_OUT)], attv)
        attg = [attv[pl.ds(cg * 16, 16)] for cg in range(8)]

        def zero_buffers():
            def zero_row(r, _):
                for j in range(C_OUT // 16):
                    msg[r, pl.ds(j * 16, 16)] = jnp.zeros((16,), jnp.float32)
                    dmsg[r, pl.ds(j * 16, 16)] = jnp.zeros((16,), jnp.float32)
                return 0
            lax.fori_loop(0, BLK, zero_row, 0)

        def zero_acc_slices():
            for kk in range(5):
                pltpu.sync_copy(msg.at[pl.ds(0, 64)],
                                ush.at[pl.ds(sid * NPTH + kk * 64, 64)])
            pltpu.sync_copy(msg.at[pl.ds(0, NDT)],
                            ush.at[pl.ds(NROWS + sid * NDT, NDT)])

        zero_buffers()
        zero_acc_slices()

        def blk_body(blk_i, macc):
            eb = base_e + blk_i * BLK
            pltpu.sync_copy(src_hbm.at[pl.ds(eb, BLK)], sidx)
            pltpu.sync_copy(dst_hbm.at[pl.ds(eb, BLK)], didx)
            for j in range(GPB):
                gsidx[pl.ds(j * 16, 16)] = sidx[pl.ds(j * 16, 16)] + hoff
                gdidx[pl.ds(j * 16, 16)] = didx[pl.ds(j * 16, 16)] + hoff
            cp1 = pltpu.async_copy(h2_hbm.at[gsidx], srows, sem1)
            cp2 = pltpu.async_copy(h2_hbm.at[gdidx], drows, sem2)
            cp1.wait()
            cp2.wait()

            mg = macc
            for g in range(GPB):
                sc = jnp.zeros((16,), jnp.float32)
                for jj in range(16):
                    e = g * 16 + jj
                    acc = jnp.zeros((16,), jnp.float32)
                    for cg in range(8):
                        v = (srows[e, pl.ds(cg * 16, 16)]
                             + drows[e, pl.ds(cg * 16, 16)])
                        acc = acc + attg[cg] * jnp.maximum(v, NEG_SLOPE_GAT * v)
                    for kk in (8, 4, 2, 1):
                        acc = acc + acc[iota16 ^ kk]
                    sc = jnp.where(iota16 == jj, acc, sc)
                sbuf[pl.ds(blk_i * BLK + g * 16, 16)] = sc
                mg = jnp.maximum(mg, sc)
            return mg

        macc = lax.fori_loop(0, NBLK, blk_body,
                             jnp.full((16,), -1e30, jnp.float32))

        for r in range(8):
            mbuf[r, :] = macc
        pltpu.sync_copy(mbuf, gsh.at[pl.ds(sid * 8, 8)])
        plsc.subcore_barrier()
        pltpu.sync_copy(gsh, gred)
        gv = gred[0, :]
        for r in range(1, NTILE):
            gv = jnp.maximum(gv, gred[r * 8, :])
        for kk in (8, 4, 2, 1):
            gv = jnp.maximum(gv, gv[iota16 ^ kk])
        gmax = gv
        plsc.subcore_barrier()

        for half in range(2):
            lo = half * NHALF

            def mb_body(blk_i, carry):
                eb = base_e + blk_i * BLK
                pltpu.sync_copy(src_hbm.at[pl.ds(eb, BLK)], sidx)
                pltpu.sync_copy(dst_hbm.at[pl.ds(eb, BLK)], gdidx)
                for j in range(GPB):
                    gsidx[pl.ds(j * 16, 16)] = sidx[pl.ds(j * 16, 16)] + hoff
                    dl = jnp.clip(gdidx[pl.ds(j * 16, 16)] - lo, 0, NHALF - 1)
                    didx[pl.ds(j * 16, 16)] = dl
                    didx2[pl.ds(j * 16, 16)] = (
                        NROWS + lax.shift_right_logical(dl, 3))
                pltpu.async_copy(h2_hbm.at[gsidx], srows, sem1).wait()
                zero16 = jnp.zeros((16,), jnp.float32)

                for g in range(GPB):
                    s16 = sbuf[pl.ds(blk_i * BLK + g * 16, 16)]
                    sv = sidx[pl.ds(g * 16, 16)]
                    dv = gdidx[pl.ds(g * 16, 16)]
                    eg = jnp.full((16,), eb + g * 16, jnp.int32) + iota16
                    valid = jnp.logical_and(
                        eg < ETOT, jnp.logical_or(sv != dv, eg >= E))
                    valid = jnp.logical_and(
                        valid, jnp.logical_and(dv >= lo, dv < lo + NHALF))
                    q16 = jnp.where(valid, jnp.exp(s16 - gmax), 0.0)
                    dloc = dv - lo
                    for jj in range(16):
                        e = g * 16 + jj
                        qsplat = jnp.full((16,), q16[jj], jnp.float32)
                        dlane = jnp.full((16,), dloc[jj] & 7, jnp.int32)
                        for cg in range(8):
                            msg[e, pl.ds(cg * 16, 16)] = (
                                srows[e, pl.ds(cg * 16, 16)] * qsplat)
                        dmsg[e, pl.ds(0, 16)] = jnp.where(
                            iota16 == dlane, qsplat, zero16)
                pltpu.sync_copy(msg, ush.at[didx], add=True)
                pltpu.sync_copy(dmsg, ush.at[didx2], add=True)
                return carry

            lax.fori_loop(0, NBLK, mb_body, 0)

            plsc.subcore_barrier()
            uoff = (head * 2 + half) * NACCH
            pltpu.sync_copy(ush.at[pl.ds(sid * NPTH, NPTH)],
                            u_hbm.at[pl.ds(uoff + sid * NPTH, NPTH)])
            pltpu.sync_copy(ush.at[pl.ds(NROWS + sid * NDT, NDT)],
                            u_hbm.at[pl.ds(uoff + NROWS + sid * NDT, NDT)])
            if half == 0:
                plsc.subcore_barrier()
                zero_buffers()
                zero_acc_slices()
                plsc.subcore_barrier()

    return k(h2, src2p, dst2p, attf)


def _nf_kernel(u_ref, d_ref, bias_ref, o_ref):
    u0 = u_ref[0]
    u1 = u_ref[1]
    d0 = d_ref[0] + 1e-16
    d1 = d_ref[1] + 1e-16
    o_ref[...] = 0.5 * (u0 / d0 + u1 / d1) + bias_ref[...]


def _bn_kernel(a_ref, g_ref, b_ref, o_ref):
    a = a_ref[...]
    m = jnp.mean(a, axis=1, keepdims=True)
    ex2 = jnp.mean(a * a, axis=1, keepdims=True)
    mc = 0.5 * (m[:C_OUT] + m[C_OUT:])
    ex2c = 0.5 * (ex2[:C_OUT] + ex2[C_OUT:])
    var = ex2c - mc * mc
    scale = lax.rsqrt(var + BN_EPS) * g_ref[...].reshape(C_OUT, 1)
    shift = b_ref[...].reshape(C_OUT, 1) - mc * scale
    scale2 = jnp.concatenate([scale, scale], axis=0)
    shift2 = jnp.concatenate([shift, shift], axis=0)
    y = a * scale2 + shift2
    o_ref[...] = jnp.maximum(y, NEG_SLOPE_ACT * y)


def _bn_tail(U, den, bias_out, gamma, beta):
    nf = pl.pallas_call(
        _nf_kernel,
        out_shape=jax.ShapeDtypeStruct((N, C_OUT), jnp.float32),
    )(U, den, bias_out.reshape(1, C_OUT))
    a = nf.reshape(B * C_OUT, DHW)
    return pl.pallas_call(
        _bn_kernel,
        out_shape=jax.ShapeDtypeStruct((B * C_OUT, DHW), jnp.float32),
    )(a, gamma.reshape(1, C_OUT), beta.reshape(1, C_OUT))


def kernel(x, edge_index, Wlin, blin, att, bias_out, gamma, beta):
    xf = jnp.transpose(x.reshape(B, C_IN, DHW), (0, 2, 1)).reshape(-1, C_IN)
    h2 = _linear(xf, Wlin, blin).reshape(HEADS * N, C_OUT)

    src = edge_index[0]
    dst = edge_index[1]
    loop = jnp.arange(N, dtype=jnp.int32)
    padz = jnp.zeros((EPAD - ETOT,), jnp.int32)
    src2p = jnp.concatenate([src, loop, padz])
    dst2p = jnp.concatenate([dst, loop, padz])

    U = _sc_edge(h2, src2p, dst2p, att.reshape(HEADS * C_OUT))
    U4 = U.reshape(HEADS, 2, NACCH, C_OUT)
    Um = U4[:, :, :NHALF, :].reshape(HEADS, 2 * NHALF, C_OUT)[:, :N, :]
    den = U4[:, :, NROWS:NROWS + NHALF // 8, :8].reshape(
        HEADS, 2 * NHALF)[:, :N].reshape(HEADS, N, 1)
    out = _bn_tail(Um, den, bias_out, gamma, beta)
    return out.reshape(B, C_OUT, D, H, W)


# speedup vs baseline: 14.7847x; 1.3554x over previous
import functools
import jax
import jax.numpy as jnp
from jax import lax
from jax.experimental import pallas as pl
from jax.experimental.pallas import tpu as pltpu
from jax.experimental.pallas import tpu_sc as plsc

B, C_IN = 2, 128
D, H, W = 10, 25, 20
C_OUT = 128
HEADS = 2
E = 160000
N = B * D * H * W
NEG_SLOPE_GAT = 0.2
NEG_SLOPE_ACT = 0.01
BN_EPS = 1e-5

DHW = D * H * W
PADW = 144
ETOT = E + N
NTILE = 16
BLK = 64
GPB = BLK // 16
NBLK = 168
EPT = NBLK * BLK
EPAD = NTILE * EPT
NPAD = 10112
NPT = NPAD // NTILE


def _lrelu(v, s):
    return jnp.maximum(v, s * v)


def _mm_kernel(x_ref, w_ref, b_ref, o_ref):
    o_ref[0] = (
        jnp.dot(x_ref[...], w_ref[...], preferred_element_type=jnp.float32)
        + b_ref[0]
    )


def _linear(xf, Wlin, blin):
    blk = 2000
    grid = (HEADS, N // blk)
    return pl.pallas_call(
        _mm_kernel,
        grid=grid,
        in_specs=[
            pl.BlockSpec((blk, C_IN), lambda h, i: (i, 0)),
            pl.BlockSpec((C_IN, C_OUT), lambda h, i: (0, h)),
            pl.BlockSpec((1, 1, C_OUT), lambda h, i: (h, 0, 0)),
        ],
        out_specs=pl.BlockSpec((1, blk, C_OUT), lambda h, i: (h, i, 0)),
        out_shape=jax.ShapeDtypeStruct((HEADS, N, C_OUT), jnp.float32),
    )(xf, Wlin, blin.reshape(HEADS, 1, C_OUT))


NPAD2 = 10112
NDEN = 640
NACC = NPAD2 + NDEN
NPT = NPAD2 // NTILE
NDT = NDEN // NTILE


def _sc_edge(h2, src2p, dst2p, attf):
    mesh = plsc.VectorSubcoreMesh(core_axis_name="c", subcore_axis_name="s")

    @functools.partial(
        pl.kernel,
        out_type=(
            jax.ShapeDtypeStruct((HEADS * NACC, C_OUT), jnp.float32),
            jax.ShapeDtypeStruct((HEADS * EPAD,), jnp.float32),
        ),
        mesh=mesh,
        scratch_types=[
            pltpu.VMEM((BLK,), jnp.int32),
            pltpu.VMEM((BLK,), jnp.int32),
            pltpu.VMEM((BLK,), jnp.int32),
            pltpu.VMEM((BLK,), jnp.int32),
            pltpu.VMEM((BLK, C_OUT), jnp.float32),
            pltpu.VMEM((BLK, C_OUT), jnp.float32),
            pltpu.VMEM((BLK, C_OUT), jnp.float32),
            pltpu.VMEM((BLK,), jnp.float32),
            pltpu.VMEM((C_OUT,), jnp.float32),
            pltpu.VMEM((8, 16), jnp.float32),
            pltpu.VMEM((NTILE * 8, 16), jnp.float32),
            pltpu.VMEM_SHARED((NACC, C_OUT), jnp.float32),
            pltpu.VMEM_SHARED((NTILE * 8, 16), jnp.float32),
            pltpu.SemaphoreType.DMA,
            pltpu.SemaphoreType.DMA,
        ],
    )
    def k(h2_hbm, src_hbm, dst_hbm, att_hbm, u_hbm, sc_hbm,
          sidx, didx, didx2, gsidx, srows, msg, dmsg, sbuf,
          attv, mbuf, gred, ush, gsh, sem1, sem2):
        head = lax.axis_index("c")
        sid = lax.axis_index("s")
        base_e = sid * EPT
        hoff = head * N
        soff = head * EPAD
        uoff = head * NACC
        iota16 = lax.iota(jnp.int32, 16)

        pltpu.sync_copy(att_hbm.at[pl.ds(head * C_OUT, C_OUT)], attv)
        attg = [attv[pl.ds(cg * 16, 16)] for cg in range(8)]

        def zero_row(r, _):
            for j in range(C_OUT // 16):
                msg[r, pl.ds(j * 16, 16)] = jnp.zeros((16,), jnp.float32)
                dmsg[r, pl.ds(j * 16, 16)] = jnp.zeros((16,), jnp.float32)
            return 0
        lax.fori_loop(0, BLK, zero_row, 0)

        for kk in range(9):
            pltpu.sync_copy(msg.at[pl.ds(0, 64)],
                            ush.at[pl.ds(sid * NPT + kk * 64, 64)])
        pltpu.sync_copy(msg.at[pl.ds(0, 56)],
                        ush.at[pl.ds(sid * NPT + 576, 56)])
        pltpu.sync_copy(msg.at[pl.ds(0, NDT)],
                        ush.at[pl.ds(NPAD2 + sid * NDT, NDT)])

        def blk_body(blk_i, macc):
            eb = base_e + blk_i * BLK
            pltpu.sync_copy(src_hbm.at[pl.ds(eb, BLK)], sidx)
            pltpu.sync_copy(dst_hbm.at[pl.ds(eb, BLK)], didx)
            for j in range(GPB):
                sidx[pl.ds(j * 16, 16)] = sidx[pl.ds(j * 16, 16)] + hoff
                didx[pl.ds(j * 16, 16)] = didx[pl.ds(j * 16, 16)] + hoff
            cp1 = pltpu.async_copy(h2_hbm.at[sidx], srows, sem1)
            cp2 = pltpu.async_copy(h2_hbm.at[didx], msg, sem2)
            cp1.wait()
            cp2.wait()

            mg = macc
            for g in range(GPB):
                sc = jnp.zeros((16,), jnp.float32)
                for jj in range(16):
                    e = g * 16 + jj
                    acc = jnp.zeros((16,), jnp.float32)
                    for cg in range(8):
                        v = (srows[e, pl.ds(cg * 16, 16)]
                             + msg[e, pl.ds(cg * 16, 16)])
                        acc = acc + attg[cg] * jnp.maximum(v, NEG_SLOPE_GAT * v)
                    for kk in (8, 4, 2, 1):
                        acc = acc + acc[iota16 ^ kk]
                    sc = jnp.where(iota16 == jj, acc, sc)
                sbuf[pl.ds(g * 16, 16)] = sc
                mg = jnp.maximum(mg, sc)
            pltpu.sync_copy(sbuf, sc_hbm.at[pl.ds(soff + eb, BLK)])
            return mg

        macc = lax.fori_loop(0, NBLK, blk_body,
                             jnp.full((16,), -1e30, jnp.float32))

        for r in range(8):
            mbuf[r, :] = macc
        pltpu.sync_copy(mbuf, gsh.at[pl.ds(sid * 8, 8)])
        plsc.subcore_barrier()
        pltpu.sync_copy(gsh, gred)
        gv = gred[0, :]
        for r in range(1, NTILE):
            gv = jnp.maximum(gv, gred[r * 8, :])
        for kk in (8, 4, 2, 1):
            gv = jnp.maximum(gv, gv[iota16 ^ kk])
        gmax = gv
        plsc.subcore_barrier()

        def mb_body(blk_i, carry):
            eb = base_e + blk_i * BLK
            pltpu.sync_copy(src_hbm.at[pl.ds(eb, BLK)], sidx)
            pltpu.sync_copy(dst_hbm.at[pl.ds(eb, BLK)], didx)
            pltpu.sync_copy(sc_hbm.at[pl.ds(soff + eb, BLK)], sbuf)
            for j in range(GPB):
                gsidx[pl.ds(j * 16, 16)] = sidx[pl.ds(j * 16, 16)] + hoff
                didx2[pl.ds(j * 16, 16)] = (
                    NPAD2 + lax.shift_right_logical(didx[pl.ds(j * 16, 16)], 4))
            pltpu.async_copy(h2_hbm.at[gsidx], srows, sem1).wait()
            zero16 = jnp.zeros((16,), jnp.float32)

            for g in range(GPB):
                s16 = sbuf[pl.ds(g * 16, 16)]
                sv = sidx[pl.ds(g * 16, 16)]
                dv = didx[pl.ds(g * 16, 16)]
                eg = jnp.full((16,), eb + g * 16, jnp.int32) + iota16
                valid = jnp.logical_and(
                    eg < ETOT, jnp.logical_or(sv != dv, eg >= E))
                q16 = jnp.where(valid, jnp.exp(s16 - gmax), 0.0)
                for jj in range(16):
                    e = g * 16 + jj
                    qsplat = jnp.full((16,), q16[jj], jnp.float32)
                    dlane = jnp.full((16,), dv[jj] & 15, jnp.int32)
                    for cg in range(8):
                        msg[e, pl.ds(cg * 16, 16)] = (
                            srows[e, pl.ds(cg * 16, 16)] * qsplat)
                    dmsg[e, pl.ds(0, 16)] = jnp.where(
                        iota16 == dlane, qsplat, zero16)
            pltpu.sync_copy(msg, ush.at[didx], add=True)
            pltpu.sync_copy(dmsg, ush.at[didx2], add=True)
            return carry

        lax.fori_loop(0, NBLK, mb_body, 0)

        plsc.subcore_barrier()
        pltpu.sync_copy(ush.at[pl.ds(sid * NPT, NPT)],
                        u_hbm.at[pl.ds(uoff + sid * NPT, NPT)])
        pltpu.sync_copy(ush.at[pl.ds(NPAD2 + sid * NDT, NDT)],
                        u_hbm.at[pl.ds(uoff + NPAD2 + sid * NDT, NDT)])

    return k(h2, src2p, dst2p, attf)[0]


def _nf_kernel(u_ref, d_ref, bias_ref, o_ref):
    u0 = u_ref[0]
    u1 = u_ref[1]
    d0 = d_ref[0] + 1e-16
    d1 = d_ref[1] + 1e-16
    o_ref[...] = 0.5 * (u0 / d0 + u1 / d1) + bias_ref[...]


def _bn_kernel(a_ref, g_ref, b_ref, o_ref):
    a = a_ref[...]
    m = jnp.mean(a, axis=1, keepdims=True)
    ex2 = jnp.mean(a * a, axis=1, keepdims=True)
    mc = 0.5 * (m[:C_OUT] + m[C_OUT:])
    ex2c = 0.5 * (ex2[:C_OUT] + ex2[C_OUT:])
    var = ex2c - mc * mc
    scale = lax.rsqrt(var + BN_EPS) * g_ref[...].reshape(C_OUT, 1)
    shift = b_ref[...].reshape(C_OUT, 1) - mc * scale
    scale2 = jnp.concatenate([scale, scale], axis=0)
    shift2 = jnp.concatenate([shift, shift], axis=0)
    y = a * scale2 + shift2
    o_ref[...] = jnp.maximum(y, NEG_SLOPE_ACT * y)


def _bn_tail(U, den, bias_out, gamma, beta):
    nf = pl.pallas_call(
        _nf_kernel,
        out_shape=jax.ShapeDtypeStruct((N, C_OUT), jnp.float32),
    )(U, den, bias_out.reshape(1, C_OUT))
    a = nf.reshape(B * C_OUT, DHW)
    return pl.pallas_call(
        _bn_kernel,
        out_shape=jax.ShapeDtypeStruct((B * C_OUT, DHW), jnp.float32),
    )(a, gamma.reshape(1, C_OUT), beta.reshape(1, C_OUT))


def kernel(x, edge_index, Wlin, blin, att, bias_out, gamma, beta):
    xf = jnp.transpose(x.reshape(B, C_IN, DHW), (0, 2, 1)).reshape(-1, C_IN)
    h2 = _linear(xf, Wlin, blin).reshape(HEADS * N, C_OUT)

    src = edge_index[0]
    dst = edge_index[1]
    loop = jnp.arange(N, dtype=jnp.int32)
    padz = jnp.zeros((EPAD - ETOT,), jnp.int32)
    src2p = jnp.concatenate([src, loop, padz])
    dst2p = jnp.concatenate([dst, loop, padz])

    U = _sc_edge(h2, src2p, dst2p, att.reshape(HEADS * C_OUT))
    U4 = U.reshape(HEADS, NACC, C_OUT)
    Um = U4[:, :N, :]
    den = U4[:, NPAD2:NPAD2 + 632, :16].reshape(HEADS, 10112)[:, :N].reshape(
        HEADS, N, 1)
    out = _bn_tail(Um, den, bias_out, gamma, beta)
    return out.reshape(B, C_OUT, D, H, W)


# speedup vs baseline: 18.1277x; 1.2261x over previous
import functools
import jax
import jax.numpy as jnp
from jax import lax
from jax.experimental import pallas as pl
from jax.experimental.pallas import tpu as pltpu
from jax.experimental.pallas import tpu_sc as plsc

B, C_IN = 2, 128
D, H, W = 10, 25, 20
C_OUT = 128
HEADS = 2
E = 160000
N = B * D * H * W
NEG_SLOPE_GAT = 0.2
NEG_SLOPE_ACT = 0.01
BN_EPS = 1e-5

DHW = D * H * W
PADW = 144
ETOT = E + N
NTILE = 16
BLK = 64
GPB = BLK // 16
NBLK = 168
SB = 512
BPS = SB // BLK
NSB = 21
EPT = NBLK * BLK
EPAD = NTILE * EPT
NPAD = 10112
NPT = NPAD // NTILE


def _lrelu(v, s):
    return jnp.maximum(v, s * v)


def _mm_kernel(x_ref, w_ref, b_ref, o_ref):
    o_ref[0] = (
        jnp.dot(x_ref[...], w_ref[...], preferred_element_type=jnp.float32)
        + b_ref[0]
    )


def _linear(xf, Wlin, blin):
    blk = 2000
    grid = (HEADS, N // blk)
    return pl.pallas_call(
        _mm_kernel,
        grid=grid,
        in_specs=[
            pl.BlockSpec((blk, C_IN), lambda h, i: (i, 0)),
            pl.BlockSpec((C_IN, C_OUT), lambda h, i: (0, h)),
            pl.BlockSpec((1, 1, C_OUT), lambda h, i: (h, 0, 0)),
        ],
        out_specs=pl.BlockSpec((1, blk, C_OUT), lambda h, i: (h, i, 0)),
        out_shape=jax.ShapeDtypeStruct((HEADS, N, C_OUT), jnp.float32),
    )(xf, Wlin, blin.reshape(HEADS, 1, C_OUT))


NPAD2 = 10112
NDEN = 640
NACC = NPAD2 + NDEN
NPT = NPAD2 // NTILE
NDT = NDEN // NTILE


def _sc_edge(h2, src2p, dst2p, attf):
    mesh = plsc.VectorSubcoreMesh(core_axis_name="c", subcore_axis_name="s")

    @functools.partial(
        pl.kernel,
        out_type=(
            jax.ShapeDtypeStruct((HEADS * NACC, C_OUT), jnp.float32),
            jax.ShapeDtypeStruct((HEADS * EPAD,), jnp.float32),
        ),
        mesh=mesh,
        scratch_types=[
            pltpu.VMEM((SB,), jnp.int32),
            pltpu.VMEM((SB,), jnp.int32),
            pltpu.VMEM((BLK,), jnp.int32),
            pltpu.VMEM((BLK,), jnp.int32),
            pltpu.VMEM((BLK,), jnp.int32),
            pltpu.VMEM((BLK, C_OUT), jnp.float32),
            pltpu.VMEM((BLK, C_OUT), jnp.float32),
            pltpu.VMEM((BLK, C_OUT), jnp.float32),
            pltpu.VMEM((SB,), jnp.float32),
            pltpu.VMEM((C_OUT,), jnp.float32),
            pltpu.VMEM((8, 16), jnp.float32),
            pltpu.VMEM((NTILE * 8, 16), jnp.float32),
            pltpu.VMEM_SHARED((NACC, C_OUT), jnp.float32),
            pltpu.VMEM_SHARED((NTILE * 8, 16), jnp.float32),
            pltpu.SemaphoreType.DMA,
            pltpu.SemaphoreType.DMA,
        ],
    )
    def k(h2_hbm, src_hbm, dst_hbm, att_hbm, u_hbm, sc_hbm,
          sidx8, didx8, didx, didx2, gsidx, srows, msg, dmsg, sbuf8,
          attv, mbuf, gred, ush, gsh, sem1, sem2):
        head = lax.axis_index("c")
        sid = lax.axis_index("s")
        base_e = sid * EPT
        hoff = head * N
        soff = head * EPAD
        uoff = head * NACC
        iota16 = lax.iota(jnp.int32, 16)

        pltpu.sync_copy(att_hbm.at[pl.ds(head * C_OUT, C_OUT)], attv)
        attg = [attv[pl.ds(cg * 16, 16)] for cg in range(8)]

        def zero_row(r, _):
            for j in range(C_OUT // 16):
                msg[r, pl.ds(j * 16, 16)] = jnp.zeros((16,), jnp.float32)
                dmsg[r, pl.ds(j * 16, 16)] = jnp.zeros((16,), jnp.float32)
            return 0
        lax.fori_loop(0, BLK, zero_row, 0)

        for kk in range(9):
            pltpu.sync_copy(msg.at[pl.ds(0, 64)],
                            ush.at[pl.ds(sid * NPT + kk * 64, 64)])
        pltpu.sync_copy(msg.at[pl.ds(0, 56)],
                        ush.at[pl.ds(sid * NPT + 576, 56)])
        pltpu.sync_copy(msg.at[pl.ds(0, NDT)],
                        ush.at[pl.ds(NPAD2 + sid * NDT, NDT)])

        def sb_body(sb_i, macc):
            ebs = base_e + sb_i * SB
            pltpu.sync_copy(src_hbm.at[pl.ds(ebs, SB)], sidx8)
            pltpu.sync_copy(dst_hbm.at[pl.ds(ebs, SB)], didx8)
            def b_body(b, mgc):
                for j in range(GPB):
                    gsidx[pl.ds(j * 16, 16)] = (
                        sidx8[pl.ds(b * BLK + j * 16, 16)] + hoff)
                    didx[pl.ds(j * 16, 16)] = (
                        didx8[pl.ds(b * BLK + j * 16, 16)] + hoff)
                cp1 = pltpu.async_copy(h2_hbm.at[gsidx], srows, sem1)
                cp2 = pltpu.async_copy(h2_hbm.at[didx], msg, sem2)
                cp1.wait()
                cp2.wait()
                for g in range(GPB):
                    sc = jnp.zeros((16,), jnp.float32)
                    for jj in range(16):
                        e = g * 16 + jj
                        acc = jnp.zeros((16,), jnp.float32)
                        for cg in range(8):
                            v = (srows[e, pl.ds(cg * 16, 16)]
                                 + msg[e, pl.ds(cg * 16, 16)])
                            acc = acc + attg[cg] * jnp.maximum(
                                v, NEG_SLOPE_GAT * v)
                        for kk in (8, 4, 2, 1):
                            acc = acc + acc[iota16 ^ kk]
                        sc = jnp.where(iota16 == jj, acc, sc)
                    sbuf8[pl.ds(b * BLK + g * 16, 16)] = sc
                    mgc = jnp.maximum(mgc, sc)
                return mgc

            mg = lax.fori_loop(0, BPS, b_body, macc)
            pltpu.sync_copy(sbuf8, sc_hbm.at[pl.ds(soff + ebs, SB)])
            return mg

        macc = lax.fori_loop(0, NSB, sb_body,
                             jnp.full((16,), -1e30, jnp.float32))

        for r in range(8):
            mbuf[r, :] = macc
        pltpu.sync_copy(mbuf, gsh.at[pl.ds(sid * 8, 8)])
        plsc.subcore_barrier()
        pltpu.sync_copy(gsh, gred)
        gv = gred[0, :]
        for r in range(1, NTILE):
            gv = jnp.maximum(gv, gred[r * 8, :])
        for kk in (8, 4, 2, 1):
            gv = jnp.maximum(gv, gv[iota16 ^ kk])
        gmax = gv
        plsc.subcore_barrier()

        def mb_body(sb_i, carry):
            ebs = base_e + sb_i * SB
            pltpu.sync_copy(src_hbm.at[pl.ds(ebs, SB)], sidx8)
            pltpu.sync_copy(dst_hbm.at[pl.ds(ebs, SB)], didx8)
            pltpu.sync_copy(sc_hbm.at[pl.ds(soff + ebs, SB)], sbuf8)
            zero16 = jnp.zeros((16,), jnp.float32)

            def b_body(b, c2):
                for j in range(GPB):
                    dvj = didx8[pl.ds(b * BLK + j * 16, 16)]
                    gsidx[pl.ds(j * 16, 16)] = (
                        sidx8[pl.ds(b * BLK + j * 16, 16)] + hoff)
                    didx[pl.ds(j * 16, 16)] = dvj
                    didx2[pl.ds(j * 16, 16)] = (
                        NPAD2 + lax.shift_right_logical(dvj, 4))
                pltpu.async_copy(h2_hbm.at[gsidx], srows, sem1).wait()
                for g in range(GPB):
                    s16 = sbuf8[pl.ds(b * BLK + g * 16, 16)]
                    sv = sidx8[pl.ds(b * BLK + g * 16, 16)]
                    dv = didx8[pl.ds(b * BLK + g * 16, 16)]
                    eg = jnp.full((16,), ebs + b * BLK + g * 16,
                                  jnp.int32) + iota16
                    valid = jnp.logical_and(
                        eg < ETOT, jnp.logical_or(sv != dv, eg >= E))
                    q16 = jnp.where(valid, jnp.exp(s16 - gmax), 0.0)
                    for jj in range(16):
                        e = g * 16 + jj
                        qsplat = jnp.full((16,), q16[jj], jnp.float32)
                        dlane = jnp.full((16,), dv[jj] & 15, jnp.int32)
                        for cg in range(8):
                            msg[e, pl.ds(cg * 16, 16)] = (
                                srows[e, pl.ds(cg * 16, 16)] * qsplat)
                        dmsg[e, pl.ds(0, 16)] = jnp.where(
                            iota16 == dlane, qsplat, zero16)
                pltpu.sync_copy(msg, ush.at[didx], add=True)
                pltpu.sync_copy(dmsg, ush.at[didx2], add=True)
                return c2

            lax.fori_loop(0, BPS, b_body, 0)
            return carry

        lax.fori_loop(0, NSB, mb_body, 0)

        plsc.subcore_barrier()
        pltpu.sync_copy(ush.at[pl.ds(sid * NPT, NPT)],
                        u_hbm.at[pl.ds(uoff + sid * NPT, NPT)])
        pltpu.sync_copy(ush.at[pl.ds(NPAD2 + sid * NDT, NDT)],
                        u_hbm.at[pl.ds(uoff + NPAD2 + sid * NDT, NDT)])

    return k(h2, src2p, dst2p, attf)[0]


def _nf_kernel(u_ref, d_ref, bias_ref, o_ref):
    u0 = u_ref[0]
    u1 = u_ref[1]
    d0 = d_ref[0] + 1e-16
    d1 = d_ref[1] + 1e-16
    o_ref[...] = 0.5 * (u0 / d0 + u1 / d1) + bias_ref[...]


def _bn_kernel(a_ref, g_ref, b_ref, o_ref):
    a = a_ref[...]
    m = jnp.mean(a, axis=1, keepdims=True)
    ex2 = jnp.mean(a * a, axis=1, keepdims=True)
    mc = 0.5 * (m[:C_OUT] + m[C_OUT:])
    ex2c = 0.5 * (ex2[:C_OUT] + ex2[C_OUT:])
    var = ex2c - mc * mc
    scale = lax.rsqrt(var + BN_EPS) * g_ref[...].reshape(C_OUT, 1)
    shift = b_ref[...].reshape(C_OUT, 1) - mc * scale
    scale2 = jnp.concatenate([scale, scale], axis=0)
    shift2 = jnp.concatenate([shift, shift], axis=0)
    y = a * scale2 + shift2
    o_ref[...] = jnp.maximum(y, NEG_SLOPE_ACT * y)


def _bn_tail(U, den, bias_out, gamma, beta):
    nf = pl.pallas_call(
        _nf_kernel,
        out_shape=jax.ShapeDtypeStruct((N, C_OUT), jnp.float32),
    )(U, den, bias_out.reshape(1, C_OUT))
    a = nf.reshape(B * C_OUT, DHW)
    return pl.pallas_call(
        _bn_kernel,
        out_shape=jax.ShapeDtypeStruct((B * C_OUT, DHW), jnp.float32),
    )(a, gamma.reshape(1, C_OUT), beta.reshape(1, C_OUT))


def kernel(x, edge_index, Wlin, blin, att, bias_out, gamma, beta):
    xf = jnp.transpose(x.reshape(B, C_IN, DHW), (0, 2, 1)).reshape(-1, C_IN)
    h2 = _linear(xf, Wlin, blin).reshape(HEADS * N, C_OUT)

    src = edge_index[0]
    dst = edge_index[1]
    loop = jnp.arange(N, dtype=jnp.int32)
    padz = jnp.zeros((EPAD - ETOT,), jnp.int32)
    src2p = jnp.concatenate([src, loop, padz])
    dst2p = jnp.concatenate([dst, loop, padz])

    U = _sc_edge(h2, src2p, dst2p, att.reshape(HEADS * C_OUT))
    U4 = U.reshape(HEADS, NACC, C_OUT)
    Um = U4[:, :N, :]
    den = U4[:, NPAD2:NPAD2 + 632, :16].reshape(HEADS, 10112)[:, :N].reshape(
        HEADS, N, 1)
    out = _bn_tail(Um, den, bias_out, gamma, beta)
    return out.reshape(B, C_OUT, D, H, W)


# speedup vs baseline: 18.5583x; 1.0238x over previous
import functools
import jax
import jax.numpy as jnp
from jax import lax
from jax.experimental import pallas as pl
from jax.experimental.pallas import tpu as pltpu
from jax.experimental.pallas import tpu_sc as plsc

B, C_IN = 2, 128
D, H, W = 10, 25, 20
C_OUT = 128
HEADS = 2
E = 160000
N = B * D * H * W
NEG_SLOPE_GAT = 0.2
NEG_SLOPE_ACT = 0.01
BN_EPS = 1e-5

DHW = D * H * W
PADW = 144
ETOT = E + N
NTILE = 16
BLK = 64
GPB = BLK // 16
NBLK = 168
SB = 512
BPS = SB // BLK
NSB = 21
EPT = NBLK * BLK
EPAD = NTILE * EPT
NPAD = 10112
NPT = NPAD // NTILE


def _lrelu(v, s):
    return jnp.maximum(v, s * v)


def _mm_kernel(x_ref, w_ref, b_ref, o_ref):
    o_ref[0] = (
        jnp.dot(x_ref[...], w_ref[...], preferred_element_type=jnp.float32)
        + b_ref[0]
    )


def _linear(xf, Wlin, blin):
    blk = 2000
    grid = (HEADS, N // blk)
    return pl.pallas_call(
        _mm_kernel,
        grid=grid,
        in_specs=[
            pl.BlockSpec((blk, C_IN), lambda h, i: (i, 0)),
            pl.BlockSpec((C_IN, C_OUT), lambda h, i: (0, h)),
            pl.BlockSpec((1, 1, C_OUT), lambda h, i: (h, 0, 0)),
        ],
        out_specs=pl.BlockSpec((1, blk, C_OUT), lambda h, i: (h, i, 0)),
        out_shape=jax.ShapeDtypeStruct((HEADS, N, C_OUT), jnp.float32),
    )(xf, Wlin, blin.reshape(HEADS, 1, C_OUT))


NPAD2 = 10112
NDEN = 640
NACC = NPAD2 + NDEN
NPT = NPAD2 // NTILE
NDT = NDEN // NTILE


def _sc_edge(h2, src2p, dst2p, attf):
    mesh = plsc.VectorSubcoreMesh(core_axis_name="c", subcore_axis_name="s")

    @functools.partial(
        pl.kernel,
        out_type=(
            jax.ShapeDtypeStruct((HEADS * NACC, C_OUT), jnp.float32),
            jax.ShapeDtypeStruct((HEADS * EPAD,), jnp.float32),
        ),
        mesh=mesh,
        scratch_types=[
            pltpu.VMEM((SB,), jnp.int32),
            pltpu.VMEM((SB,), jnp.int32),
            pltpu.VMEM((BLK,), jnp.int32),
            pltpu.VMEM((BLK,), jnp.int32),
            pltpu.VMEM((BLK,), jnp.int32),
            pltpu.VMEM((BLK, C_OUT), jnp.float32),
            pltpu.VMEM((BLK, C_OUT), jnp.float32),
            pltpu.VMEM((BLK, C_OUT), jnp.float32),
            pltpu.VMEM((SB,), jnp.float32),
            pltpu.VMEM((C_OUT,), jnp.float32),
            pltpu.VMEM((8, 16), jnp.float32),
            pltpu.VMEM((NTILE * 8, 16), jnp.float32),
            pltpu.VMEM_SHARED((NACC, C_OUT), jnp.float32),
            pltpu.VMEM_SHARED((NTILE * 8, 16), jnp.float32),
            pltpu.SemaphoreType.DMA,
            pltpu.SemaphoreType.DMA,
            pltpu.SemaphoreType.DMA,
        ],
    )
    def k(h2_hbm, src_hbm, dst_hbm, att_hbm, u_hbm, sc_hbm,
          sidx8, didx8, didx, didx2, gsidx, srows, msg, dmsg, sbuf8,
          attv, mbuf, gred, ush, gsh, sem1, sem2, sem3):
        head = lax.axis_index("c")
        sid = lax.axis_index("s")
        base_e = sid * EPT
        hoff = head * N
        soff = head * EPAD
        uoff = head * NACC
        iota16 = lax.iota(jnp.int32, 16)

        pltpu.sync_copy(att_hbm.at[pl.ds(head * C_OUT, C_OUT)], attv)
        attg = [attv[pl.ds(cg * 16, 16)] for cg in range(8)]

        def zero_row(r, _):
            for j in range(C_OUT // 16):
                msg[r, pl.ds(j * 16, 16)] = jnp.zeros((16,), jnp.float32)
                dmsg[r, pl.ds(j * 16, 16)] = jnp.zeros((16,), jnp.float32)
            return 0
        lax.fori_loop(0, BLK, zero_row, 0)

        for kk in range(9):
            pltpu.sync_copy(msg.at[pl.ds(0, 64)],
                            ush.at[pl.ds(sid * NPT + kk * 64, 64)])
        pltpu.sync_copy(msg.at[pl.ds(0, 56)],
                        ush.at[pl.ds(sid * NPT + 576, 56)])
        pltpu.sync_copy(msg.at[pl.ds(0, NDT)],
                        ush.at[pl.ds(NPAD2 + sid * NDT, NDT)])

        def sb_body(sb_i, macc):
            ebs = base_e + sb_i * SB
            cpa = pltpu.async_copy(src_hbm.at[pl.ds(ebs, SB)], sidx8, sem1)
            cpb = pltpu.async_copy(dst_hbm.at[pl.ds(ebs, SB)], didx8, sem2)
            cpa.wait()
            cpb.wait()

            def b_body(b, mgc):
                for j in range(GPB):
                    gsidx[pl.ds(j * 16, 16)] = (
                        sidx8[pl.ds(b * BLK + j * 16, 16)] + hoff)
                    didx[pl.ds(j * 16, 16)] = (
                        didx8[pl.ds(b * BLK + j * 16, 16)] + hoff)
                cp1 = pltpu.async_copy(h2_hbm.at[gsidx], srows, sem1)
                cp2 = pltpu.async_copy(h2_hbm.at[didx], msg, sem2)
                cp1.wait()
                cp2.wait()
                for g in range(GPB):
                    sc = jnp.zeros((16,), jnp.float32)
                    for jj in range(16):
                        e = g * 16 + jj
                        acc = jnp.zeros((16,), jnp.float32)
                        for cg in range(8):
                            v = (srows[e, pl.ds(cg * 16, 16)]
                                 + msg[e, pl.ds(cg * 16, 16)])
                            acc = acc + attg[cg] * jnp.maximum(
                                v, NEG_SLOPE_GAT * v)
                        for kk in (8, 4, 2, 1):
                            acc = acc + acc[iota16 ^ kk]
                        sc = jnp.where(iota16 == jj, acc, sc)
                    sbuf8[pl.ds(b * BLK + g * 16, 16)] = sc
                    mgc = jnp.maximum(mgc, sc)
                return mgc

            mg = lax.fori_loop(0, BPS, b_body, macc)
            pltpu.sync_copy(sbuf8, sc_hbm.at[pl.ds(soff + ebs, SB)])
            return mg

        macc = lax.fori_loop(0, NSB, sb_body,
                             jnp.full((16,), -1e30, jnp.float32))

        for r in range(8):
            mbuf[r, :] = macc
        pltpu.sync_copy(mbuf, gsh.at[pl.ds(sid * 8, 8)])
        plsc.subcore_barrier()
        pltpu.sync_copy(gsh, gred)
        gv = gred[0, :]
        for r in range(1, NTILE):
            gv = jnp.maximum(gv, gred[r * 8, :])
        for kk in (8, 4, 2, 1):
            gv = jnp.maximum(gv, gv[iota16 ^ kk])
        gmax = gv
        plsc.subcore_barrier()

        def mb_body(sb_i, carry):
            ebs = base_e + sb_i * SB
            cpa = pltpu.async_copy(src_hbm.at[pl.ds(ebs, SB)], sidx8, sem1)
            cpb = pltpu.async_copy(dst_hbm.at[pl.ds(ebs, SB)], didx8, sem2)
            cpc = pltpu.async_copy(sc_hbm.at[pl.ds(soff + ebs, SB)], sbuf8, sem3)
            cpa.wait()
            cpb.wait()
            cpc.wait()
            zero16 = jnp.zeros((16,), jnp.float32)

            def b_body(b, c2):
                for j in range(GPB):
                    dvj = didx8[pl.ds(b * BLK + j * 16, 16)]
                    gsidx[pl.ds(j * 16, 16)] = (
                        sidx8[pl.ds(b * BLK + j * 16, 16)] + hoff)
                    didx[pl.ds(j * 16, 16)] = dvj
                    didx2[pl.ds(j * 16, 16)] = (
                        NPAD2 + lax.shift_right_logical(dvj, 4))
                pltpu.async_copy(h2_hbm.at[gsidx], srows, sem1).wait()
                for g in range(GPB):
                    s16 = sbuf8[pl.ds(b * BLK + g * 16, 16)]
                    sv = sidx8[pl.ds(b * BLK + g * 16, 16)]
                    dv = didx8[pl.ds(b * BLK + g * 16, 16)]
                    eg = jnp.full((16,), ebs + b * BLK + g * 16,
                                  jnp.int32) + iota16
                    valid = jnp.logical_and(
                        eg < ETOT, jnp.logical_or(sv != dv, eg >= E))
                    q16 = jnp.where(valid, jnp.exp(s16 - gmax), 0.0)
                    for jj in range(16):
                        e = g * 16 + jj
                        qsplat = jnp.full((16,), q16[jj], jnp.float32)
                        dlane = jnp.full((16,), dv[jj] & 15, jnp.int32)
                        for cg in range(8):
                            msg[e, pl.ds(cg * 16, 16)] = (
                                srows[e, pl.ds(cg * 16, 16)] * qsplat)
                        dmsg[e, pl.ds(0, 16)] = jnp.where(
                            iota16 == dlane, qsplat, zero16)
                pltpu.sync_copy(msg, ush.at[didx], add=True)
                pltpu.sync_copy(dmsg, ush.at[didx2], add=True)
                return c2

            lax.fori_loop(0, BPS, b_body, 0)
            return carry

        lax.fori_loop(0, NSB, mb_body, 0)

        plsc.subcore_barrier()
        pltpu.sync_copy(ush.at[pl.ds(sid * NPT, NPT)],
                        u_hbm.at[pl.ds(uoff + sid * NPT, NPT)])
        pltpu.sync_copy(ush.at[pl.ds(NPAD2 + sid * NDT, NDT)],
                        u_hbm.at[pl.ds(uoff + NPAD2 + sid * NDT, NDT)])

    return k(h2, src2p, dst2p, attf)[0]


def _nf_kernel(u_ref, d_ref, bias_ref, o_ref):
    u0 = u_ref[0]
    u1 = u_ref[1]
    d0 = d_ref[0] + 1e-16
    d1 = d_ref[1] + 1e-16
    o_ref[...] = 0.5 * (u0 / d0 + u1 / d1) + bias_ref[...]


def _bn_kernel(a_ref, g_ref, b_ref, o_ref):
    a = a_ref[...]
    m = jnp.mean(a, axis=1, keepdims=True)
    ex2 = jnp.mean(a * a, axis=1, keepdims=True)
    mc = 0.5 * (m[:C_OUT] + m[C_OUT:])
    ex2c = 0.5 * (ex2[:C_OUT] + ex2[C_OUT:])
    var = ex2c - mc * mc
    scale = lax.rsqrt(var + BN_EPS) * g_ref[...].reshape(C_OUT, 1)
    shift = b_ref[...].reshape(C_OUT, 1) - mc * scale
    scale2 = jnp.concatenate([scale, scale], axis=0)
    shift2 = jnp.concatenate([shift, shift], axis=0)
    y = a * scale2 + shift2
    o_ref[...] = jnp.maximum(y, NEG_SLOPE_ACT * y)


def _bn_tail(U, den, bias_out, gamma, beta):
    nf = pl.pallas_call(
        _nf_kernel,
        out_shape=jax.ShapeDtypeStruct((N, C_OUT), jnp.float32),
    )(U, den, bias_out.reshape(1, C_OUT))
    a = nf.reshape(B * C_OUT, DHW)
    return pl.pallas_call(
        _bn_kernel,
        out_shape=jax.ShapeDtypeStruct((B * C_OUT, DHW), jnp.float32),
    )(a, gamma.reshape(1, C_OUT), beta.reshape(1, C_OUT))


def kernel(x, edge_index, Wlin, blin, att, bias_out, gamma, beta):
    xf = jnp.transpose(x.reshape(B, C_IN, DHW), (0, 2, 1)).reshape(-1, C_IN)
    h2 = _linear(xf, Wlin, blin).reshape(HEADS * N, C_OUT)

    src = edge_index[0]
    dst = edge_index[1]
    loop = jnp.arange(N, dtype=jnp.int32)
    padz = jnp.zeros((EPAD - ETOT,), jnp.int32)
    src2p = jnp.concatenate([src, loop, padz])
    dst2p = jnp.concatenate([dst, loop, padz])

    U = _sc_edge(h2, src2p, dst2p, att.reshape(HEADS * C_OUT))
    U4 = U.reshape(HEADS, NACC, C_OUT)
    Um = U4[:, :N, :]
    den = U4[:, NPAD2:NPAD2 + 632, :16].reshape(HEADS, 10112)[:, :N].reshape(
        HEADS, N, 1)
    out = _bn_tail(Um, den, bias_out, gamma, beta)
    return out.reshape(B, C_OUT, D, H, W)
